# Initial kernel scaffold; baseline (speedup 1.0000x reference)
import jax
import jax.numpy as jnp
from jax.experimental import pallas as pl


def kernel(tokens, values, table, W_val, b_val, tok_g, tok_b, val_g, val_b, fin_g, fin_b):
    raise NotImplementedError("write your pallas kernel here")


# speedup vs baseline: 1.8554x; 1.8554x over previous
import functools

import jax
import jax.numpy as jnp
from jax import lax
from jax.experimental import pallas as pl
from jax.experimental.pallas import tpu as pltpu
from jax.experimental.pallas import tpu_sc as plsc

HID = 64
EPS = 1e-5
SCALE = 8.0

NC = 2
NS = 16
NW = NC * NS

CHUNK = 512


def _sc_gather(table128, idx, n_rows):
    per_w = n_rows // NW
    n_chunks = per_w // CHUNK
    mesh = plsc.VectorSubcoreMesh(core_axis_name="c", subcore_axis_name="s")

    @functools.partial(
        pl.kernel,
        out_type=jax.ShapeDtypeStruct((n_rows, 128), jnp.float32),
        mesh=mesh,
        scratch_types=[
            pltpu.VMEM((CHUNK,), jnp.int32),
            pltpu.VMEM((CHUNK, 128), jnp.float32),
            pltpu.SemaphoreType.DMA,
        ],
    )
    def gather_kernel(table_hbm, idx_hbm, out_hbm, idx_v, rows_v, sem):
        wid = lax.axis_index("s") * NC + lax.axis_index("c")
        base = wid * per_w

        def body(i, carry):
            r0 = base + i * CHUNK
            pltpu.sync_copy(idx_hbm.at[pl.ds(r0, CHUNK)], idx_v)
            pltpu.async_copy(table_hbm.at[idx_v], rows_v, sem).wait()
            pltpu.sync_copy(rows_v, out_hbm.at[pl.ds(r0, CHUNK)])
            return carry

        lax.fori_loop(0, n_chunks, body, 0, unroll=False)

    return gather_kernel(table128, idx)


def _tc_math_body(g_ref, v_ref, t_ref, wv_ref, bv_ref, tg_ref, tb_ref,
                  vg_ref, vb_ref, fg_ref, fb_ref, out_ref, m_ref):
    g = g_ref[:, :HID]
    mu = jnp.mean(g, axis=-1, keepdims=True)
    gc = g - mu
    var = jnp.mean(gc * gc, axis=-1, keepdims=True)
    t = gc * lax.rsqrt(var + EPS) * tg_ref[...] + tb_ref[...]

    v = v_ref[...]
    e = v * wv_ref[...] + bv_ref[...]
    mu2 = jnp.mean(e, axis=-1, keepdims=True)
    ec = e - mu2
    var2 = jnp.mean(ec * ec, axis=-1, keepdims=True)
    u = ec * lax.rsqrt(var2 + EPS) * vg_ref[...] + vb_ref[...]

    y = (t + u) * SCALE
    mu3 = jnp.mean(y, axis=-1, keepdims=True)
    yc = y - mu3
    var3 = jnp.mean(yc * yc, axis=-1, keepdims=True)
    out_ref[...] = yc * lax.rsqrt(var3 + EPS) * fg_ref[...] + fb_ref[...]
    m_ref[...] = (t_ref[...] != 0).astype(jnp.int8)


def _tc_math(gathered, values2d, tokens2d, W_val, b_val, tok_g, tok_b,
             val_g, val_b, fin_g, fin_b):
    n = gathered.shape[0]
    R = 2048
    grid = (n // R,)
    gat_spec = pl.BlockSpec((R, 128), lambda i: (i, 0))
    row_spec = pl.BlockSpec((R, HID), lambda i: (i, 0))
    col_spec = pl.BlockSpec((R, 1), lambda i: (i, 0))
    par_spec = pl.BlockSpec((1, HID), lambda i: (0, 0))
    return pl.pallas_call(
        _tc_math_body,
        grid=grid,
        in_specs=[gat_spec, col_spec, col_spec] + [par_spec] * 8,
        out_specs=[row_spec, col_spec],
        out_shape=[
            jax.ShapeDtypeStruct((n, HID), jnp.float32),
            jax.ShapeDtypeStruct((n, 1), jnp.int8),
        ],
    )(gathered, values2d, tokens2d,
      W_val.reshape(1, HID), b_val.reshape(1, HID),
      tok_g.reshape(1, HID), tok_b.reshape(1, HID),
      val_g.reshape(1, HID), val_b.reshape(1, HID),
      fin_g.reshape(1, HID), fin_b.reshape(1, HID))


def kernel(tokens, values, table, W_val, b_val, tok_g, tok_b, val_g, val_b, fin_g, fin_b):
    B, L = tokens.shape
    n = B * L
    idx = tokens.reshape(n).astype(jnp.int32)
    table128 = jnp.pad(table, ((0, 0), (0, 128 - HID)))
    gathered = _sc_gather(table128, idx, n)
    emb, mask = _tc_math(gathered, values.reshape(n, 1),
                         tokens.reshape(n, 1).astype(jnp.int32),
                         W_val, b_val, tok_g, tok_b, val_g, val_b,
                         fin_g, fin_b)
    return emb.reshape(B, L, HID), mask.reshape(B, L).astype(jnp.bool_)


# speedup vs baseline: 2.1243x; 1.1449x over previous
import functools

import jax
import jax.numpy as jnp
from jax import lax
from jax.experimental import pallas as pl
from jax.experimental.pallas import tpu as pltpu
from jax.experimental.pallas import tpu_sc as plsc

HID = 64
EPS = 1e-5
SCALE = 8.0

NC = 2
NS = 16
NW = NC * NS

CHUNK = 512
VLANE = 67


def _sc_gather(table128, idx, values, n_rows):
    per_w = n_rows // NW
    n_chunks = per_w // CHUNK
    mesh = plsc.VectorSubcoreMesh(core_axis_name="c", subcore_axis_name="s")

    @functools.partial(
        pl.kernel,
        out_type=jax.ShapeDtypeStruct((n_rows, 128), jnp.float32),
        mesh=mesh,
        scratch_types=[
            pltpu.VMEM((CHUNK,), jnp.int32),
            pltpu.VMEM((CHUNK,), jnp.float32),
            pltpu.VMEM((CHUNK, 128), jnp.float32),
            pltpu.SemaphoreType.DMA,
        ],
    )
    def gather_kernel(table_hbm, idx_hbm, val_hbm, out_hbm,
                      idx_v, val_v, rows_v, sem):
        wid = lax.axis_index("s") * NC + lax.axis_index("c")
        base = wid * per_w

        def body(i, carry):
            r0 = base + i * CHUNK
            pltpu.sync_copy(idx_hbm.at[pl.ds(r0, CHUNK)], idx_v)
            pltpu.sync_copy(val_hbm.at[pl.ds(r0, CHUNK)], val_v)
            pltpu.async_copy(table_hbm.at[idx_v], rows_v, sem).wait()
            pos = lax.iota(jnp.int32, 16)

            def merge(k, c2):
                vv = val_v[pl.ds(16 * k, 16)]
                for j in range(16):
                    vj = jnp.where(pos == 0, vv[j], 0.0)
                    rows_v[16 * k + j, pl.ds(VLANE, 16)] = vj
                return c2

            lax.fori_loop(0, CHUNK // 16, merge, 0, unroll=False)
            pltpu.sync_copy(rows_v, out_hbm.at[pl.ds(r0, CHUNK)])
            return carry

        lax.fori_loop(0, n_chunks, body, 0, unroll=False)

    return gather_kernel(table128, idx, values)


def _tc_math_body(g_ref, t_ref, wv_ref, bv_ref, tg_ref, tb_ref,
                  vg_ref, vb_ref, fg_ref, fb_ref, out_ref, m_ref):
    g3 = g_ref[...]
    c = g3[:, :, :HID]
    v = g3[:, :, VLANE:VLANE + 1]

    mu = jnp.mean(c, axis=-1, keepdims=True)
    cc = c - mu
    var = jnp.mean(cc * cc, axis=-1, keepdims=True)
    t = cc * lax.rsqrt(var + EPS) * tg_ref[...] + tb_ref[...]

    e = v * wv_ref[...] + bv_ref[...]
    mu2 = jnp.mean(e, axis=-1, keepdims=True)
    ec = e - mu2
    var2 = jnp.mean(ec * ec, axis=-1, keepdims=True)
    u = ec * lax.rsqrt(var2 + EPS) * vg_ref[...] + vb_ref[...]

    y = (t + u) * SCALE
    mu3 = jnp.mean(y, axis=-1, keepdims=True)
    yc = y - mu3
    var3 = jnp.mean(yc * yc, axis=-1, keepdims=True)
    out_ref[...] = yc * lax.rsqrt(var3 + EPS) * fg_ref[...] + fb_ref[...]
    m_ref[...] = (t_ref[...] != 0).astype(jnp.int8)


def _tc_math(g3d, tokens, W_val, b_val, tok_g, tok_b, val_g, val_b,
             fin_g, fin_b):
    B, L, _ = g3d.shape
    BB = 8
    grid = (B // BB,)
    g_spec = pl.BlockSpec((BB, L, 128), lambda i: (i, 0, 0))
    tok_spec = pl.BlockSpec((BB, L), lambda i: (i, 0))
    out_spec = pl.BlockSpec((BB, L, HID), lambda i: (i, 0, 0))
    par_spec = pl.BlockSpec((1, 1, HID), lambda i: (0, 0, 0))
    return pl.pallas_call(
        _tc_math_body,
        grid=grid,
        in_specs=[g_spec, tok_spec] + [par_spec] * 8,
        out_specs=[out_spec, tok_spec],
        out_shape=[
            jax.ShapeDtypeStruct((B, L, HID), jnp.float32),
            jax.ShapeDtypeStruct((B, L), jnp.int8),
        ],
    )(g3d, tokens,
      W_val.reshape(1, 1, HID), b_val.reshape(1, 1, HID),
      tok_g.reshape(1, 1, HID), tok_b.reshape(1, 1, HID),
      val_g.reshape(1, 1, HID), val_b.reshape(1, 1, HID),
      fin_g.reshape(1, 1, HID), fin_b.reshape(1, 1, HID))


def kernel(tokens, values, table, W_val, b_val, tok_g, tok_b, val_g, val_b, fin_g, fin_b):
    B, L = tokens.shape
    n = B * L
    idx = tokens.reshape(n).astype(jnp.int32)
    table128 = jnp.pad(table, ((0, 0), (0, 128 - HID)))
    gathered = _sc_gather(table128, idx, values.reshape(n), n)
    g3d = gathered.reshape(B, L, 128)
    emb, mask = _tc_math(g3d, tokens.astype(jnp.int32), W_val, b_val,
                         tok_g, tok_b, val_g, val_b, fin_g, fin_b)
    return emb, mask.astype(jnp.bool_)
